# baseline (speedup 1.0000x reference)
import functools

import jax
import jax.numpy as jnp
from jax import lax
from jax.experimental import pallas as pl
from jax.experimental.pallas import tpu as pltpu
from jax.experimental.pallas import tpu_sc as plsc

HID = 128
B = 1024
S = 200
SPAD = 208
MAXPOS = 256
NA = 104
NB = 96
LANES = 16
NCHUNK = SPAD // LANES

_NC = 2
_NS = 16
NW = _NC * _NS
ROWS_PER_W = B // NW


def _sc_gather(ids_flat, word, pos_adj):
    mesh = plsc.VectorSubcoreMesh(core_axis_name="c", subcore_axis_name="s")
    TILE_TOK = ROWS_PER_W * S

    @functools.partial(
        pl.kernel,
        out_type=jax.ShapeDtypeStruct((B * S, HID), jnp.float32),
        mesh=mesh,
        scratch_types=[
            pltpu.VMEM((TILE_TOK + LANES,), jnp.int32),
            pltpu.VMEM((MAXPOS, HID), jnp.float32),
            pltpu.VMEM((S, HID), jnp.float32),
            pltpu.VMEM((S, HID), jnp.float32),
            pltpu.VMEM((SPAD,), jnp.int32),
            pltpu.VMEM((SPAD,), jnp.int32),
            pltpu.SemaphoreType.DMA,
            pltpu.SemaphoreType.DMA,
            pltpu.SemaphoreType.DMA,
            pltpu.SemaphoreType.DMA,
        ],
        compiler_params=pltpu.CompilerParams(needs_layout_passes=False),
    )
    def k(ids_hbm, word_hbm, pos_hbm, out_hbm, bigids, ptab, wbuf0, wbuf1,
          pbuf0, pbuf1, gsem0, gsem1, wsem0, wsem1):
        wid = lax.axis_index("s") * _NC + lax.axis_index("c")
        tbase = wid * TILE_TOK
        pltpu.sync_copy(ids_hbm.at[pl.ds(tbase, TILE_TOK)],
                        bigids.at[pl.ds(0, TILE_TOK)])
        pltpu.sync_copy(pos_hbm, ptab)

        wbufs = (wbuf0, wbuf1)
        pbufs = (pbuf0, pbuf1)
        gsems = (gsem0, gsem1)
        wsems = (wsem0, wsem1)

        lane = lax.iota(jnp.int32, LANES)
        ntail = S - (SPAD - LANES)
        tailmask = lax.shift_right_logical(
            (ntail - 1) - lane + 16, jnp.int32(4)
        ) & 1
        tailmask_f = tailmask.astype(jnp.float32)
        toks = [g * LANES + lane for g in range(NCHUNK - 1)]
        toks.append(jnp.minimum((NCHUNK - 1) * LANES + lane, S - 1))

        def fire(i, p):
            ib = i * S
            pltpu.async_copy(word_hbm.at[bigids.at[pl.ds(ib, NA)]],
                             wbufs[p].at[pl.ds(0, NA)], gsems[p])
            pltpu.async_copy(word_hbm.at[bigids.at[pl.ds(ib + NA, NB)]],
                             wbufs[p].at[pl.ds(NA, NB)], gsems[p])
            carry = jnp.int32(0)
            for c in range(NCHUNK):
                v = bigids[pl.ds(ib + c * LANES, LANES)]
                if c == NCHUNK - 1:
                    v = v * tailmask
                m = jnp.minimum(jnp.abs(v), 1)
                cs = plsc.cumsum(m)
                pbufs[p][pl.ds(c * LANES, LANES)] = (cs + carry) * m
                carry = carry + jnp.sum(m)

        def wait_gather(i, p):
            ib = i * S
            pltpu.make_async_copy(word_hbm.at[bigids.at[pl.ds(ib, NA)]],
                                  wbufs[p].at[pl.ds(0, NA)], gsems[p]).wait()
            pltpu.make_async_copy(word_hbm.at[bigids.at[pl.ds(ib + NA, NB)]],
                                  wbufs[p].at[pl.ds(NA, NB)], gsems[p]).wait()

        def pos_add(p):
            wbuf = wbufs[p]
            pids = [pbufs[p][pl.ds(g * LANES, LANES)] for g in range(NCHUNK)]

            def jc_body(jc, cc):
                for jj in range(LANES):
                    jvec = jnp.zeros((LANES,), jnp.int32) + (jc * LANES + jj)
                    for g in range(NCHUNK):
                        vals = plsc.load_gather(ptab, [pids[g], jvec])
                        if g == NCHUNK - 1:
                            vals = vals * tailmask_f
                        plsc.addupdate_scatter(wbuf, [toks[g], jvec], vals)
                return cc

            lax.fori_loop(0, HID // LANES, jc_body, 0)

        def fire_wb(i, p):
            pltpu.async_copy(wbufs[p], out_hbm.at[pl.ds(tbase + i * S, S)],
                             wsems[p])

        def wait_wb(i, p):
            pltpu.make_async_copy(wbufs[p],
                                  out_hbm.at[pl.ds(tbase + i * S, S)],
                                  wsems[p]).wait()

        fire(0, 0)
        fire(1, 1)

        def pair_body(h, c0):
            e = 2 * h
            wait_gather(e, 0)
            pos_add(0)
            fire_wb(e, 0)
            wait_gather(e + 1, 1)
            pos_add(1)
            fire_wb(e + 1, 1)
            wait_wb(e, 0)
            fire(e + 2, 0)
            wait_wb(e + 1, 1)
            fire(e + 3, 1)
            return c0

        lax.fori_loop(0, ROWS_PER_W // 2 - 1, pair_body, 0)
        last = ROWS_PER_W - 2
        wait_gather(last, 0)
        pos_add(0)
        fire_wb(last, 0)
        wait_gather(last + 1, 1)
        pos_add(1)
        fire_wb(last + 1, 1)
        wait_wb(last, 0)
        wait_wb(last + 1, 1)

    return k(ids_flat, word, pos_adj)


def _tc_ln(x_rows, gamma, beta):
    ROWS = 4096
    n_blocks = (B * S) // ROWS
    INV = 1.0 / HID

    def body(x_ref, g_ref, b_ref, o_ref):
        x = x_ref[...]
        c = lax.broadcasted_iota(jnp.int32, (HID, HID), 1)
        w = jnp.where(c == 0, INV, 0.0).astype(jnp.bfloat16)
        xb = x.astype(jnp.bfloat16)
        mu = jax.lax.dot_general(
            xb, w, (((1,), (0,)), ((), ())),
            preferred_element_type=jnp.float32)[:, 0:1]
        m2 = jax.lax.dot_general(
            xb * xb, w, (((1,), (0,)), ((), ())),
            preferred_element_type=jnp.float32)[:, 0:1]
        var = jnp.maximum(m2 - mu * mu, 0.0)
        o_ref[...] = (x - mu) * lax.rsqrt(var + 1e-12) * g_ref[...] + b_ref[...]

    return pl.pallas_call(
        body,
        grid=(n_blocks,),
        in_specs=[
            pl.BlockSpec((ROWS, HID), lambda i: (i, 0)),
            pl.BlockSpec((1, HID), lambda i: (0, 0)),
            pl.BlockSpec((1, HID), lambda i: (0, 0)),
        ],
        out_specs=pl.BlockSpec((ROWS, HID), lambda i: (i, 0)),
        out_shape=jax.ShapeDtypeStruct((B * S, HID), jnp.float32),
    )(x_rows, gamma, beta)


def kernel(input_ids, word_embeddings, position_embeddings,
           token_type_embeddings, ln_weight, ln_bias):
    ids = input_ids.astype(jnp.int32)
    pos_adj = position_embeddings + token_type_embeddings[0:1]
    summed = _sc_gather(ids.reshape(B * S), word_embeddings, pos_adj)
    out = _tc_ln(summed, ln_weight.reshape(1, HID), ln_bias.reshape(1, HID))
    return out.reshape(B, S, HID)


# speedup vs baseline: 5.8249x; 5.8249x over previous
import functools

import jax
import jax.numpy as jnp
from jax import lax
from jax.experimental import pallas as pl
from jax.experimental.pallas import tpu as pltpu
from jax.experimental.pallas import tpu_sc as plsc

HID = 128
B = 1024
S = 200
SPAD = 208
MAXPOS = 256
NA = 104
NB = 96
LANES = 16

_NC = 2
_NS = 16
NW = _NC * _NS
ROWS_PER_W = B // NW


def _sc_gather(ids_flat, word):
    mesh = plsc.VectorSubcoreMesh(core_axis_name="c", subcore_axis_name="s")
    TILE_TOK = ROWS_PER_W * S

    @functools.partial(
        pl.kernel,
        out_type=(jax.ShapeDtypeStruct((B * S, HID), jnp.float32),
                  jax.ShapeDtypeStruct((B * S,), jnp.int32)),
        mesh=mesh,
        scratch_types=[
            pltpu.VMEM((TILE_TOK + LANES,), jnp.int32),
            pltpu.VMEM((S, HID), jnp.float32),
            pltpu.VMEM((S, HID), jnp.float32),
            pltpu.VMEM((SPAD,), jnp.int32),
            pltpu.VMEM((SPAD,), jnp.int32),
            pltpu.SemaphoreType.DMA,
            pltpu.SemaphoreType.DMA,
            pltpu.SemaphoreType.DMA,
            pltpu.SemaphoreType.DMA,
        ],
        compiler_params=pltpu.CompilerParams(needs_layout_passes=False),
    )
    def k(ids_hbm, word_hbm, out_hbm, pid_hbm, bigids, wbuf0, wbuf1,
          pbuf0, pbuf1, gsem0, gsem1, wsem0, wsem1):
        wid = lax.axis_index("s") * _NC + lax.axis_index("c")
        tbase = wid * TILE_TOK
        pltpu.sync_copy(ids_hbm.at[pl.ds(tbase, TILE_TOK)],
                        bigids.at[pl.ds(0, TILE_TOK)])

        wbufs = (wbuf0, wbuf1)
        pbufs = (pbuf0, pbuf1)
        gsems = (gsem0, gsem1)
        wsems = (wsem0, wsem1)

        lane = lax.iota(jnp.int32, LANES)
        ntail = S - (SPAD - LANES)
        tailmask = lax.shift_right_logical(
            (ntail - 1) - lane + 16, jnp.int32(4)
        ) & 1

        def fire(i, p):
            ib = i * S
            pltpu.async_copy(word_hbm.at[bigids.at[pl.ds(ib, NA)]],
                             wbufs[p].at[pl.ds(0, NA)], gsems[p])
            pltpu.async_copy(word_hbm.at[bigids.at[pl.ds(ib + NA, NB)]],
                             wbufs[p].at[pl.ds(NA, NB)], gsems[p])
            carry = jnp.int32(0)
            for c in range(SPAD // LANES):
                v = bigids[pl.ds(ib + c * LANES, LANES)]
                if c == SPAD // LANES - 1:
                    v = v * tailmask
                m = jnp.minimum(jnp.abs(v), 1)
                cs = plsc.cumsum(m)
                pbufs[p][pl.ds(c * LANES, LANES)] = (cs + carry) * m
                carry = carry + jnp.sum(m)

        def wait_gather(i, p):
            ib = i * S
            pltpu.make_async_copy(word_hbm.at[bigids.at[pl.ds(ib, NA)]],
                                  wbufs[p].at[pl.ds(0, NA)], gsems[p]).wait()
            pltpu.make_async_copy(word_hbm.at[bigids.at[pl.ds(ib + NA, NB)]],
                                  wbufs[p].at[pl.ds(NA, NB)], gsems[p]).wait()

        def fire_wb(i, p):
            pltpu.async_copy(wbufs[p], out_hbm.at[pl.ds(tbase + i * S, S)],
                             wsems[p])
            pltpu.async_copy(pbufs[p].at[pl.ds(0, S)],
                             pid_hbm.at[pl.ds(tbase + i * S, S)], wsems[p])

        def wait_wb(i, p):
            pltpu.make_async_copy(wbufs[p],
                                  out_hbm.at[pl.ds(tbase + i * S, S)],
                                  wsems[p]).wait()
            pltpu.make_async_copy(pbufs[p].at[pl.ds(0, S)],
                                  pid_hbm.at[pl.ds(tbase + i * S, S)],
                                  wsems[p]).wait()

        fire(0, 0)
        fire(1, 1)

        def pair_body(h, c0):
            e = 2 * h
            wait_gather(e, 0)
            fire_wb(e, 0)
            wait_gather(e + 1, 1)
            fire_wb(e + 1, 1)
            wait_wb(e, 0)
            fire(e + 2, 0)
            wait_wb(e + 1, 1)
            fire(e + 3, 1)
            return c0

        lax.fori_loop(0, ROWS_PER_W // 2 - 1, pair_body, 0)
        last = ROWS_PER_W - 2
        wait_gather(last, 0)
        fire_wb(last, 0)
        wait_gather(last + 1, 1)
        fire_wb(last + 1, 1)
        wait_wb(last, 0)
        wait_wb(last + 1, 1)

    return k(ids_flat, word)


def _tc_posln(word_rows, posid3, pos_adj, gamma, beta):
    ROWS = 4096
    n_blocks = (B * S) // ROWS
    INV = 1.0 / HID

    def body(x_ref, pid_ref, pos_ref, g_ref, b_ref, o_ref):
        pid = pid_ref[0]
        prow = lax.broadcasted_iota(jnp.int32, (MAXPOS, 1), 0)
        onehot_t = jnp.where(prow == pid, 1.0, 0.0).astype(jnp.bfloat16)
        pos_emb = jax.lax.dot_general(
            onehot_t, pos_ref[...],
            (((0,), (0,)), ((), ())),
            preferred_element_type=jnp.float32)

        x = x_ref[...] + pos_emb
        w_all = jnp.full((HID, HID), INV, jnp.bfloat16)
        xb = x.astype(jnp.bfloat16)
        mu_b = jax.lax.dot_general(
            xb, w_all, (((1,), (0,)), ((), ())),
            preferred_element_type=jnp.float32)
        m2 = jax.lax.dot_general(
            xb * xb, w_all, (((1,), (0,)), ((), ())),
            preferred_element_type=jnp.float32)[:, 0:1]
        mu = mu_b[:, 0:1]
        var = jnp.maximum(m2 - mu * mu, 0.0)
        rstd = lax.rsqrt(var + 1e-12)
        o_ref[...] = (x - mu_b) * rstd * g_ref[...] + b_ref[...]

    return pl.pallas_call(
        body,
        grid=(n_blocks,),
        in_specs=[
            pl.BlockSpec((ROWS, HID), lambda i: (i, 0)),
            pl.BlockSpec((1, 1, ROWS), lambda i: (i, 0, 0)),
            pl.BlockSpec((MAXPOS, HID), lambda i: (0, 0)),
            pl.BlockSpec((1, HID), lambda i: (0, 0)),
            pl.BlockSpec((1, HID), lambda i: (0, 0)),
        ],
        out_specs=pl.BlockSpec((ROWS, HID), lambda i: (i, 0)),
        out_shape=jax.ShapeDtypeStruct((B * S, HID), jnp.float32),
    )(word_rows, posid3, pos_adj, gamma, beta)


def kernel(input_ids, word_embeddings, position_embeddings,
           token_type_embeddings, ln_weight, ln_bias):
    ids = input_ids.astype(jnp.int32)
    word_rows, posid = _sc_gather(ids.reshape(B * S), word_embeddings)
    ROWS = 4096
    posid3 = posid.reshape((B * S) // ROWS, 1, ROWS)
    pos_adj = (position_embeddings
               + token_type_embeddings[0:1]).astype(jnp.bfloat16)
    out = _tc_posln(word_rows, posid3, pos_adj,
                    ln_weight.reshape(1, HID), ln_bias.reshape(1, HID))
    return out.reshape(B, S, HID)
